# Initial kernel scaffold; baseline (speedup 1.0000x reference)
import jax
import jax.numpy as jnp
from jax.experimental import pallas as pl


def kernel(inputs, table):
    raise NotImplementedError("write your pallas kernel here")


# speedup vs baseline: 4.6078x; 4.6078x over previous
import functools

import jax
import jax.numpy as jnp
from jax import lax
from jax.experimental import pallas as pl
from jax.experimental.pallas import tpu as pltpu
from jax.experimental.pallas import tpu_sc as plsc

B, L, D = 16384, 200, 64
N = B * L
SCALE = 8.0

NC, NS = 2, 16
NW = NC * NS
PER_W = N // NW

IDX_COLS = 128
CHUNK_ROWS = 4
GROUP = CHUNK_ROWS * IDX_COLS
GROUPS = PER_W // GROUP

_PRE_ROWS = 1000


def _prescale_body(table_ref, out_ref):
    i = pl.program_id(0)
    row = lax.broadcasted_iota(jnp.int32, table_ref.shape, 0) + i * _PRE_ROWS
    out_ref[...] = jnp.where(row == 0, 0.0, table_ref[...] * SCALE)


def _prescale(table):
    v, d = table.shape
    return pl.pallas_call(
        _prescale_body,
        grid=(v // _PRE_ROWS,),
        in_specs=[pl.BlockSpec((_PRE_ROWS, d), lambda i: (i, 0))],
        out_specs=pl.BlockSpec((_PRE_ROWS, d), lambda i: (i, 0)),
        out_shape=jax.ShapeDtypeStruct((v, d), jnp.float32),
    )(table)


_mesh = plsc.VectorSubcoreMesh(core_axis_name="c", subcore_axis_name="s")


@functools.partial(
    pl.kernel,
    mesh=_mesh,
    out_type=jax.ShapeDtypeStruct((N, D), jnp.float32),
    scratch_types=[
        pltpu.VMEM((CHUNK_ROWS, IDX_COLS), jnp.int32),
        pltpu.VMEM((GROUP, D), jnp.float32),
        pltpu.SemaphoreType.DMA,
    ],
    compiler_params=pltpu.CompilerParams(use_tc_tiling_on_sc=False),
)
def _gather(table_hbm, idx_hbm, out_hbm, idx_v, rows_v, sem):
    wid = lax.axis_index("s") * NC + lax.axis_index("c")
    base = wid * PER_W
    base_row = wid * (PER_W // IDX_COLS)

    def body(g, carry):
        pltpu.sync_copy(
            idx_hbm.at[pl.ds(base_row + g * CHUNK_ROWS, CHUNK_ROWS)], idx_v
        )
        copies = [
            pltpu.async_copy(
                table_hbm.at[idx_v.at[j]],
                rows_v.at[pl.ds(j * IDX_COLS, IDX_COLS)],
                sem,
            )
            for j in range(CHUNK_ROWS)
        ]
        for cp in copies:
            cp.wait()
        pltpu.sync_copy(rows_v, out_hbm.at[pl.ds(base + g * GROUP, GROUP)])
        return carry

    lax.fori_loop(0, GROUPS, body, 0)


def kernel(inputs, table):
    scaled = _prescale(table.astype(jnp.float32))
    idx2d = inputs.reshape(N // IDX_COLS, IDX_COLS).astype(jnp.int32)
    out_flat = _gather(scaled, idx2d)
    return out_flat.reshape(B, L, D)


# speedup vs baseline: 4.8923x; 1.0618x over previous
import functools

import jax
import jax.numpy as jnp
from jax import lax
from jax.experimental import pallas as pl
from jax.experimental.pallas import tpu as pltpu
from jax.experimental.pallas import tpu_sc as plsc

B, L, D = 16384, 200, 64
N = B * L
SCALE = 8.0

NC, NS = 2, 16
NW = NC * NS
PER_W = N // NW

IDX_COLS = 128
CHUNK_ROWS = 4
GROUP = CHUNK_ROWS * IDX_COLS
GROUPS = PER_W // GROUP

_PRE_ROWS = 1000


def _prescale_body(table_ref, out_ref):
    i = pl.program_id(0)
    row = lax.broadcasted_iota(jnp.int32, table_ref.shape, 0) + i * _PRE_ROWS
    out_ref[...] = jnp.where(row == 0, 0.0, table_ref[...] * SCALE)


def _prescale(table):
    v, d = table.shape
    return pl.pallas_call(
        _prescale_body,
        grid=(v // _PRE_ROWS,),
        in_specs=[pl.BlockSpec((_PRE_ROWS, d), lambda i: (i, 0))],
        out_specs=pl.BlockSpec((_PRE_ROWS, d), lambda i: (i, 0)),
        out_shape=jax.ShapeDtypeStruct((v, d), jnp.float32),
    )(table)


_mesh = plsc.VectorSubcoreMesh(core_axis_name="c", subcore_axis_name="s")

NBUF = 2


@functools.partial(
    pl.kernel,
    mesh=_mesh,
    out_type=jax.ShapeDtypeStruct((N, D), jnp.float32),
    scratch_types=[
        [pltpu.VMEM((CHUNK_ROWS, IDX_COLS), jnp.int32) for _ in range(NBUF)],
        [pltpu.VMEM((GROUP, D), jnp.float32) for _ in range(NBUF)],
        [pltpu.SemaphoreType.DMA for _ in range(NBUF)],
        [pltpu.SemaphoreType.DMA for _ in range(NBUF)],
    ],
    compiler_params=pltpu.CompilerParams(use_tc_tiling_on_sc=False),
)
def _gather(table_hbm, idx_hbm, out_hbm, idx_bufs, row_bufs, gsems, ssems):
    wid = lax.axis_index("s") * NC + lax.axis_index("c")
    base = wid * PER_W
    base_row = wid * (PER_W // IDX_COLS)

    def fire_gather(g, b):
        pltpu.sync_copy(
            idx_hbm.at[pl.ds(base_row + g * CHUNK_ROWS, CHUNK_ROWS)], idx_bufs[b]
        )
        for j in range(CHUNK_ROWS):
            pltpu.async_copy(
                table_hbm.at[idx_bufs[b].at[j]],
                row_bufs[b].at[pl.ds(j * IDX_COLS, IDX_COLS)],
                gsems[b],
            )

    def wait_gather(b):
        for j in range(CHUNK_ROWS):
            pltpu.make_async_copy(
                table_hbm.at[idx_bufs[b].at[j]],
                row_bufs[b].at[pl.ds(j * IDX_COLS, IDX_COLS)],
                gsems[b],
            ).wait()

    for b in range(NBUF):
        fire_gather(b, b)

    def body(t, carry):
        for b in range(NBUF):
            g = t * NBUF + b
            wait_gather(b)
            st = pltpu.async_copy(
                row_bufs[b], out_hbm.at[pl.ds(base + g * GROUP, GROUP)], ssems[b]
            )
            st.wait()

            @pl.when(g + NBUF < GROUPS)
            def _():
                fire_gather(g + NBUF, b)

        return carry

    lax.fori_loop(0, GROUPS // NBUF, body, 0)


def kernel(inputs, table):
    scaled = _prescale(table.astype(jnp.float32))
    idx2d = inputs.reshape(N // IDX_COLS, IDX_COLS).astype(jnp.int32)
    out_flat = _gather(scaled, idx2d)
    return out_flat.reshape(B, L, D)
